# Initial kernel scaffold; baseline (speedup 1.0000x reference)
import jax
import jax.numpy as jnp
from jax.experimental import pallas as pl


def kernel(x, tables):
    raise NotImplementedError("write your pallas kernel here")


# speedup vs baseline: 1.9009x; 1.9009x over previous
import functools

import jax
import jax.numpy as jnp
from jax import lax
from jax.experimental import pallas as pl
from jax.experimental.pallas import tpu as pltpu
from jax.experimental.pallas import tpu_sc as plsc

_NC = 2
_NS = 16
_L = 16
_NW = _NC * _NS

_B = 128


def _sc_body(x3_hbm, tab_hbm, out_hbm, tab_v, xb_v, out_v, *, F, V, D, nb):
    wid = lax.axis_index("s") * _NC + lax.axis_index("c")
    K = D // 2
    pltpu.sync_copy(tab_hbm, tab_v)
    riota = lax.iota(jnp.int32, _L) * D

    def batch_body(k, carry):
        bidx = wid * nb + k
        pltpu.sync_copy(x3_hbm.at[bidx], xb_v)
        for g in range(_B // _L):
            base = []
            for i in range(F):
                base.append((xb_v[i, pl.ds(g * _L, _L)] + i * V) * K)
            out_off = riota + g * _L * D

            @plsc.parallel_loop(0, K, step=1, unroll=4)
            def dbody(c, base=base, out_off=out_off):
                g2 = [plsc.bitcast(plsc.load_gather(tab_v, [base[i] + c]),
                                   jnp.bfloat16)
                      for i in range(F)]
                while len(g2) > 1:
                    g2 = [g2[j] + g2[j + 1] for j in range(0, len(g2) - 1, 2)] \
                        + ([g2[-1]] if len(g2) % 2 else [])
                lo, hi = plsc.unpack(g2[0], format=plsc.PackFormat.INTERLEAVED)
                plsc.store_scatter(out_v, [out_off + 2 * c], lo)
                plsc.store_scatter(out_v, [out_off + 2 * c + 1], hi)
        pltpu.sync_copy(out_v, out_hbm.at[pl.ds(bidx * (_B * D), _B * D)])
        return carry

    lax.fori_loop(0, nb, batch_body, 0, unroll=False)


def kernel(x, tables):
    N, F = x.shape
    _, V, D = tables.shape
    nb = -(-N // (_NW * _B))
    Np = _NW * _B * nb

    xpad = jnp.pad(x, ((0, Np - N), (0, 0)))
    x3 = xpad.reshape(Np // _B, _B, F).transpose(0, 2, 1)
    tab_pack = lax.bitcast_convert_type(
        tables.astype(jnp.bfloat16).reshape(F * V * D // 2, 2), jnp.int32)

    mesh = plsc.VectorSubcoreMesh(core_axis_name="c", subcore_axis_name="s")
    body = functools.partial(_sc_body, F=F, V=V, D=D, nb=nb)
    out_flat = pl.kernel(
        body,
        out_type=jax.ShapeDtypeStruct((Np * D,), jnp.float32),
        mesh=mesh,
        scratch_types=[
            pltpu.VMEM((F * V * D // 2,), jnp.int32),
            pltpu.VMEM((F, _B), jnp.int32),
            pltpu.VMEM((_B * D,), jnp.float32),
        ],
        compiler_params=pltpu.CompilerParams(needs_layout_passes=False),
    )(x3, tab_pack)
    return out_flat.reshape(Np, D)[:N]


# speedup vs baseline: 3.1601x; 1.6624x over previous
import functools

import jax
import jax.numpy as jnp
from jax import lax
from jax.experimental import pallas as pl
from jax.experimental.pallas import tpu as pltpu
from jax.experimental.pallas import tpu_sc as plsc

_NC = 2
_NS = 16
_L = 16
_NW = _NC * _NS

_B = 128


def _sc_body(x3_hbm, tab_hbm, out_hbm, tab_v, out_v, xb_v, *, F, V, D, nb):
    wid = lax.axis_index("s") * _NC + lax.axis_index("c")
    K = D // 2
    pltpu.sync_copy(tab_hbm, tab_v)

    def batch_body(k, carry):
        bidx = wid * nb + k
        pltpu.sync_copy(x3_hbm.at[bidx], xb_v)

        def group_body(gi, c2):
            base_vs = [(xb_v[pl.ds(i * _B + gi * _L, _L)] + i * V) * K
                       for i in range(F)]
            for b in range(_L):
                rows = [base_vs[i][b] for i in range(F)]
                out_row = gi * (_L * K) + b * K
                for c in range(K // _L):
                    g = [plsc.bitcast(tab_v[pl.ds(rows[i] + c * _L, _L)],
                                      jnp.bfloat16)
                         for i in range(F)]
                    while len(g) > 1:
                        g = [g[j] + g[j + 1]
                             for j in range(0, len(g) - 1, 2)] \
                            + ([g[-1]] if len(g) % 2 else [])
                    out_v[pl.ds(out_row + c * _L, _L)] = plsc.bitcast(
                        g[0], jnp.int32)
            return c2

        lax.fori_loop(0, _B // _L, group_body, 0, unroll=False)
        pltpu.sync_copy(out_v, out_hbm.at[pl.ds(bidx * (_B * K), _B * K)])
        return carry

    lax.fori_loop(0, nb, batch_body, 0, unroll=False)


def kernel(x, tables):
    N, F = x.shape
    _, V, D = tables.shape
    K = D // 2
    nb = -(-N // (_NW * _B))
    Np = _NW * _B * nb

    xpad = jnp.pad(x, ((0, Np - N), (0, 0)))
    x3 = xpad.reshape(Np // _B, _B, F).transpose(0, 2, 1).reshape(
        Np // _B, F * _B)
    tab_pack = lax.bitcast_convert_type(
        tables.astype(jnp.bfloat16).reshape(F * V * K, 2), jnp.int32)

    mesh = plsc.VectorSubcoreMesh(core_axis_name="c", subcore_axis_name="s")
    body = functools.partial(_sc_body, F=F, V=V, D=D, nb=nb)
    out_pack = pl.kernel(
        body,
        out_type=jax.ShapeDtypeStruct((Np * K,), jnp.int32),
        mesh=mesh,
        scratch_types=[
            pltpu.VMEM((F * V * K,), jnp.int32),
            pltpu.VMEM((_B * K,), jnp.int32),
            pltpu.VMEM((F * _B,), jnp.int32),
        ],
        compiler_params=pltpu.CompilerParams(needs_layout_passes=False),
    )(x3, tab_pack)
    out_bf = lax.bitcast_convert_type(out_pack.reshape(Np, K), jnp.bfloat16)
    return out_bf.reshape(Np, D).astype(jnp.float32)[:N]


# speedup vs baseline: 5.3688x; 1.6989x over previous
import functools

import jax
import jax.numpy as jnp
from jax import lax
from jax.experimental import pallas as pl
from jax.experimental.pallas import tpu as pltpu
from jax.experimental.pallas import tpu_sc as plsc

_NC = 2
_NS = 16
_L = 16
_NW = _NC * _NS

_B = 128


def _sc_body(x_hbm, tab_hbm, out_hbm, tab_v, out_v, xb_v, *, F, V, D, N):
    wid = lax.axis_index("s") * _NC + lax.axis_index("c")
    K = D // 2
    rows_w = N // _NW
    nb = -(-rows_w // _B)
    xlen = ((_B * F + 7 + _L + 7) // 8) * 8
    pltpu.sync_copy(tab_hbm, tab_v)

    def batch_body(j, carry):
        base = wid * rows_w + jnp.minimum(j * _B, rows_w - _B)
        xoff = base * F
        xstart = pl.multiple_of((xoff >> 3) << 3, 8)
        delta = xoff - xstart
        pltpu.sync_copy(x_hbm.at[pl.ds(xstart, xlen)], xb_v)

        def group_body(gi, c2):
            goff = delta + gi * (_L * F)
            for b in range(_L):
                xv = xb_v[pl.ds(goff + b * F, _L)]
                rows = [(xv[i] + i * V) * K for i in range(F)]
                out_row = gi * (_L * D) + b * D
                for c in range(K // _L):
                    g = [plsc.bitcast(tab_v[pl.ds(rows[i] + c * _L, _L)],
                                      jnp.bfloat16)
                         for i in range(F)]
                    while len(g) > 1:
                        g = [g[j2] + g[j2 + 1]
                             for j2 in range(0, len(g) - 1, 2)] \
                            + ([g[-1]] if len(g) % 2 else [])
                    lo, hi = plsc.unpack(g[0],
                                         format=plsc.PackFormat.INTERLEAVED)
                    out_v[pl.ds(out_row + c * 2 * _L, _L)] = lo
                    out_v[pl.ds(out_row + c * 2 * _L + _L, _L)] = hi
            return c2

        lax.fori_loop(0, _B // _L, group_body, 0, unroll=False)
        pltpu.sync_copy(
            out_v, out_hbm.at[pl.ds(pl.multiple_of(base * D, 8), _B * D)])
        return carry

    lax.fori_loop(0, nb, batch_body, 0, unroll=False)


def kernel(x, tables):
    N, F = x.shape
    _, V, D = tables.shape
    K = D // 2

    xpad = ((_B * F + 7 + _L + 7) // 8) * 8
    x_flat = jnp.pad(x.reshape(N * F), (0, xpad))
    tab_pairs = lax.bitcast_convert_type(
        tables.astype(jnp.bfloat16).reshape(F * V, D // 32, 2, _L)
        .transpose(0, 1, 3, 2).reshape(F * V * K, 2), jnp.int32)

    mesh = plsc.VectorSubcoreMesh(core_axis_name="c", subcore_axis_name="s")
    body = functools.partial(_sc_body, F=F, V=V, D=D, N=N)
    out = pl.kernel(
        body,
        out_type=jax.ShapeDtypeStruct((N * D,), jnp.float32),
        mesh=mesh,
        scratch_types=[
            pltpu.VMEM((F * V * K,), jnp.int32),
            pltpu.VMEM((_B * D,), jnp.float32),
            pltpu.VMEM((((_B * F + 7 + _L + 7) // 8) * 8,), jnp.int32),
        ],
        compiler_params=pltpu.CompilerParams(needs_layout_passes=False),
    )(x_flat, tab_pairs)
    return out.reshape(N, D)
